# baseline (speedup 1.0000x reference)
import functools

import jax
import jax.numpy as jnp
from jax import lax
from jax.experimental import pallas as pl
from jax.experimental.pallas import tpu as pltpu
from jax.experimental.pallas import tpu_sc as plsc

B, S, D = 4, 4096, 2048
N = B * S
NW = 32
SEG = N // NW
C = 16
NCH = SEG // C
TMAX = NCH + 1


def _prep(attack_mask):
    m = attack_mask.reshape(NW, SEG).astype(jnp.int32)
    cs1 = jnp.cumsum(m, axis=1)
    cs0 = jnp.cumsum(1 - m, axis=1)
    nxw = cs0[:, -1:]
    rank = jnp.where(m == 0, cs0 - 1, nxw + cs1 - 1)
    rows = jnp.broadcast_to(jnp.arange(SEG, dtype=jnp.int32)[None, :],
                            (NW, SEG))
    perm = jnp.zeros((NW, SEG), jnp.int32).at[
        jnp.arange(NW)[:, None], rank].set(rows)
    permg = perm + (jnp.arange(NW, dtype=jnp.int32) * SEG)[:, None]
    p = jnp.arange(SEG, dtype=jnp.int32)[None, :]
    ix = jnp.clip(p, 0, jnp.maximum(nxw - 1, 0))
    ia = jnp.clip(jnp.maximum(p, nxw), 0, SEG - 1)
    ent_x = jnp.take_along_axis(permg, ix, axis=1).reshape(-1)
    ent_a = jnp.take_along_axis(permg, ia, axis=1).reshape(-1)
    meta = nxw.reshape(2, 16)
    return ent_x, ent_a, meta


def _sc_body(entx, enta, meta, xf, af, out,
             meta_v, ibuf, buf, msem, isem, gsem, ssem):
    c = lax.axis_index("c")
    s = lax.axis_index("s")
    wid = c * 16 + s
    mcp = pltpu.make_async_copy(meta.at[c], meta_v, msem)
    mcp.start()
    mcp.wait()
    mv = meta_v[...]
    nxw = jnp.int32(0)
    for k in range(16):
        nxw = nxw + jnp.where(s == k, mv[k], 0)
    n_xc = (nxw + C - 1) // C
    t = n_xc + (NCH - nxw // C)
    base = wid * SEG

    def job(i, carry):
        @pl.when(i < t)
        def _():
            is_x = i < n_xc
            chunk = jnp.where(is_x, i, nxw // C + (i - n_xc))
            off = base + chunk * C

            @pl.when(is_x)
            def _():
                cp = pltpu.make_async_copy(entx.at[pl.ds(off, C)], ibuf, isem)
                cp.start()
                cp.wait()
                g = pltpu.make_async_copy(xf.at[ibuf], buf, gsem)
                g.start()
                g.wait()

            @pl.when(jnp.logical_not(is_x))
            def _():
                cp = pltpu.make_async_copy(enta.at[pl.ds(off, C)], ibuf, isem)
                cp.start()
                cp.wait()
                g = pltpu.make_async_copy(af.at[ibuf], buf, gsem)
                g.start()
                g.wait()

            sc = pltpu.make_async_copy(buf, out.at[ibuf], ssem)
            sc.start()
            sc.wait()
        return carry

    lax.fori_loop(0, TMAX, job, 0)


@jax.jit
def _sc_select(xf, af, ent_x, ent_a, meta):
    mesh = plsc.VectorSubcoreMesh(core_axis_name="c", subcore_axis_name="s")
    kern = functools.partial(
        pl.kernel,
        out_type=jax.ShapeDtypeStruct((N, D), jnp.float32),
        mesh=mesh,
        scratch_types=[
            pltpu.VMEM((16,), jnp.int32),
            pltpu.VMEM((C,), jnp.int32),
            pltpu.VMEM((C, D), jnp.float32),
            pltpu.SemaphoreType.DMA,
            pltpu.SemaphoreType.DMA,
            pltpu.SemaphoreType.DMA,
            pltpu.SemaphoreType.DMA,
        ],
    )(_sc_body)
    return kern(ent_x, ent_a, meta, xf, af)


def kernel(x, attack, attack_mask):
    xf = x.reshape(N, D)
    af = attack.reshape(N, D)
    ent_x, ent_a, meta = _prep(attack_mask)
    out = _sc_select(xf, af, ent_x, ent_a, meta)
    return out.reshape(B, S, D)


# speedup vs baseline: 1.1465x; 1.1465x over previous
import functools

import jax
import jax.numpy as jnp
from jax import lax
from jax.experimental import pallas as pl
from jax.experimental.pallas import tpu as pltpu
from jax.experimental.pallas import tpu_sc as plsc

B, S, D = 4, 4096, 2048
N = B * S
NW = 32
SEG = N // NW
C = 16
NCH = SEG // C
TMAX = NCH + 1


def _prep(attack_mask):
    m = attack_mask.reshape(NW, SEG).astype(jnp.int32)
    cs1 = jnp.cumsum(m, axis=1)
    cs0 = jnp.cumsum(1 - m, axis=1)
    nxw = cs0[:, -1:]
    rank = jnp.where(m == 0, cs0 - 1, nxw + cs1 - 1)
    rows = jnp.broadcast_to(jnp.arange(SEG, dtype=jnp.int32)[None, :],
                            (NW, SEG))
    perm = jnp.zeros((NW, SEG), jnp.int32).at[
        jnp.arange(NW)[:, None], rank].set(rows)
    permg = perm + (jnp.arange(NW, dtype=jnp.int32) * SEG)[:, None]
    p = jnp.arange(SEG, dtype=jnp.int32)[None, :]
    ix = jnp.clip(p, 0, jnp.maximum(nxw - 1, 0))
    ia = jnp.clip(jnp.maximum(p, nxw), 0, SEG - 1)
    ent_x = jnp.take_along_axis(permg, ix, axis=1).reshape(-1)
    ent_a = jnp.take_along_axis(permg, ia, axis=1).reshape(-1)
    meta = nxw.reshape(2, 16)
    return ent_x, ent_a, meta


NBANK = 3
NGRP = (TMAX + NBANK - 1) // NBANK


def _sc_body(entx, enta, meta, xf, af, out,
             meta_v, ib0, ib1, ib2, b0, b1, b2,
             msem, is0, is1, is2, gs0, gs1, gs2, ss0, ss1, ss2):
    ibufs = (ib0, ib1, ib2)
    bufs = (b0, b1, b2)
    isems = (is0, is1, is2)
    gsems = (gs0, gs1, gs2)
    ssems = (ss0, ss1, ss2)

    c = lax.axis_index("c")
    s = lax.axis_index("s")
    wid = c * 16 + s
    mcp = pltpu.make_async_copy(meta.at[c], meta_v, msem)
    mcp.start()
    mcp.wait()
    mv = meta_v[...]
    nxw = jnp.int32(0)
    for k in range(16):
        nxw = nxw + jnp.where(s == k, mv[k], 0)
    n_xc = (nxw + C - 1) // C
    t = n_xc + (NCH - nxw // C)
    base = wid * SEG

    def ent_off(i):
        is_x = i < n_xc
        chunk = jnp.where(is_x, i, nxw // C + (i - n_xc))
        return is_x, base + chunk * C

    def start_idx(i, bank):
        is_x, off = ent_off(i)

        @pl.when(is_x)
        def _():
            pltpu.make_async_copy(
                entx.at[pl.ds(off, C)], ibufs[bank], isems[bank]).start()

        @pl.when(jnp.logical_not(is_x))
        def _():
            pltpu.make_async_copy(
                enta.at[pl.ds(off, C)], ibufs[bank], isems[bank]).start()

    start_idx(0, 0)

    def group(g, carry):
        for b in range(NBANK):
            i = g * NBANK + b

            @pl.when(i < t)
            def _(i=i, b=b):
                nb = (b + 1) % NBANK
                pltpu.make_async_copy(
                    entx.at[pl.ds(0, C)], ibufs[b], isems[b]).wait()

                @pl.when(i + 1 < t)
                def _():
                    @pl.when(i >= 2)
                    def _():
                        pltpu.make_async_copy(
                            bufs[nb], out.at[ibufs[nb]], ssems[nb]).wait()
                    start_idx(i + 1, nb)

                is_x, _ = ent_off(i)

                @pl.when(is_x)
                def _():
                    pltpu.make_async_copy(
                        xf.at[ibufs[b]], bufs[b], gsems[b]).start()

                @pl.when(jnp.logical_not(is_x))
                def _():
                    pltpu.make_async_copy(
                        af.at[ibufs[b]], bufs[b], gsems[b]).start()

                pltpu.make_async_copy(
                    xf.at[ibufs[b]], bufs[b], gsems[b]).wait()

                pltpu.make_async_copy(
                    bufs[b], out.at[ibufs[b]], ssems[b]).start()
        return carry

    lax.fori_loop(0, NGRP, group, 0)

    for b in range(NBANK):
        pltpu.make_async_copy(bufs[b], out.at[ibufs[b]], ssems[b]).wait()


@jax.jit
def _sc_select(xf, af, ent_x, ent_a, meta):
    mesh = plsc.VectorSubcoreMesh(core_axis_name="c", subcore_axis_name="s")
    kern = functools.partial(
        pl.kernel,
        out_type=jax.ShapeDtypeStruct((N, D), jnp.float32),
        mesh=mesh,
        scratch_types=(
            [pltpu.VMEM((16,), jnp.int32)]
            + [pltpu.VMEM((C,), jnp.int32) for _ in range(NBANK)]
            + [pltpu.VMEM((C, D), jnp.float32) for _ in range(NBANK)]
            + [pltpu.SemaphoreType.DMA for _ in range(1 + 3 * NBANK)]
        ),
    )(_sc_body)
    return kern(ent_x, ent_a, meta, xf, af)


def kernel(x, attack, attack_mask):
    xf = x.reshape(N, D)
    af = attack.reshape(N, D)
    ent_x, ent_a, meta = _prep(attack_mask)
    out = _sc_select(xf, af, ent_x, ent_a, meta)
    return out.reshape(B, S, D)


# speedup vs baseline: 1.2669x; 1.1050x over previous
import functools

import jax
import jax.numpy as jnp
from jax import lax
from jax.experimental import pallas as pl
from jax.experimental.pallas import tpu as pltpu
from jax.experimental.pallas import tpu_sc as plsc

B, S, D = 4, 4096, 2048
N = B * S
NW = 32
SEG = N // NW
C = 16
NCH = SEG // C
TMAX = NCH + 1


def _prep(attack_mask):
    m = attack_mask.reshape(NW, SEG).astype(jnp.int32)
    cs1 = jnp.cumsum(m, axis=1)
    p = jnp.arange(SEG, dtype=jnp.int32)[None, :]
    cs0 = (p + 1) - cs1
    nxw = cs0[:, -1:]
    rank = jnp.where(m == 0, cs0 - 1, nxw + cs1 - 1)
    rows = jnp.broadcast_to(p, (NW, SEG))
    permg = jnp.zeros((NW, SEG), jnp.int32).at[
        jnp.arange(NW)[:, None], rank].set(
        rows + (jnp.arange(NW, dtype=jnp.int32) * SEG)[:, None])
    lastx = jnp.take_along_axis(permg, jnp.maximum(nxw - 1, 0), axis=1)
    firsta = jnp.take_along_axis(permg, jnp.minimum(nxw, SEG - 1), axis=1)
    ent_x = jnp.where(p >= nxw, lastx, permg).reshape(-1)
    ent_a = jnp.where(p < nxw, firsta, permg).reshape(-1)
    meta = nxw.reshape(2, 16)
    return ent_x, ent_a, meta


NBANK = 3
NGRP = (TMAX + NBANK - 1) // NBANK


def _sc_body(entx, enta, meta, xf, af, out,
             meta_v, ib0, ib1, ib2, b0, b1, b2,
             msem, is0, is1, is2, gs0, gs1, gs2, ss0, ss1, ss2):
    ibufs = (ib0, ib1, ib2)
    bufs = (b0, b1, b2)
    isems = (is0, is1, is2)
    gsems = (gs0, gs1, gs2)
    ssems = (ss0, ss1, ss2)

    c = lax.axis_index("c")
    s = lax.axis_index("s")
    wid = c * 16 + s
    mcp = pltpu.make_async_copy(meta.at[c], meta_v, msem)
    mcp.start()
    mcp.wait()
    mv = meta_v[...]
    nxw = jnp.int32(0)
    for k in range(16):
        nxw = nxw + jnp.where(s == k, mv[k], 0)
    n_xc = (nxw + C - 1) // C
    t = n_xc + (NCH - nxw // C)
    base = wid * SEG

    def ent_off(i):
        is_x = i < n_xc
        chunk = jnp.where(is_x, i, nxw // C + (i - n_xc))
        return is_x, base + chunk * C

    def start_idx(i, bank):
        is_x, off = ent_off(i)

        @pl.when(is_x)
        def _():
            pltpu.make_async_copy(
                entx.at[pl.ds(off, C)], ibufs[bank], isems[bank]).start()

        @pl.when(jnp.logical_not(is_x))
        def _():
            pltpu.make_async_copy(
                enta.at[pl.ds(off, C)], ibufs[bank], isems[bank]).start()

    start_idx(0, 0)

    def group(g, carry):
        for b in range(NBANK):
            i = g * NBANK + b
            nb = (b + 1) % NBANK
            pb = (b + 2) % NBANK

            @pl.when(i < t)
            def _(i=i, b=b, nb=nb, pb=pb):
                pltpu.make_async_copy(
                    entx.at[pl.ds(0, C)], ibufs[b], isems[b]).wait()

                @pl.when(i >= 2)
                def _():
                    pltpu.make_async_copy(
                        bufs[nb], out.at[ibufs[nb]], ssems[nb]).wait()

                @pl.when(i + 1 < t)
                def _():
                    start_idx(i + 1, nb)

                is_x, _ = ent_off(i)

                @pl.when(is_x)
                def _():
                    pltpu.make_async_copy(
                        xf.at[ibufs[b]], bufs[b], gsems[b]).start()

                @pl.when(jnp.logical_not(is_x))
                def _():
                    pltpu.make_async_copy(
                        af.at[ibufs[b]], bufs[b], gsems[b]).start()

                @pl.when(i >= 1)
                def _():
                    pltpu.make_async_copy(
                        xf.at[ibufs[pb]], bufs[pb], gsems[pb]).wait()
                    pltpu.make_async_copy(
                        bufs[pb], out.at[ibufs[pb]], ssems[pb]).start()
        return carry

    lax.fori_loop(0, NGRP, group, 0)

    for b in range(NBANK):
        @pl.when((t - 1) % NBANK == b)
        def _(b=b):
            pltpu.make_async_copy(xf.at[ibufs[b]], bufs[b], gsems[b]).wait()
            pltpu.make_async_copy(bufs[b], out.at[ibufs[b]], ssems[b]).start()
    for b in range(NBANK):
        @pl.when(jnp.logical_or((t - 1) % NBANK == b, (t - 2) % NBANK == b))
        def _(b=b):
            pltpu.make_async_copy(bufs[b], out.at[ibufs[b]], ssems[b]).wait()


@jax.jit
def _sc_select(xf, af, ent_x, ent_a, meta):
    mesh = plsc.VectorSubcoreMesh(core_axis_name="c", subcore_axis_name="s")
    kern = functools.partial(
        pl.kernel,
        out_type=jax.ShapeDtypeStruct((N, D), jnp.float32),
        mesh=mesh,
        scratch_types=(
            [pltpu.VMEM((16,), jnp.int32)]
            + [pltpu.VMEM((C,), jnp.int32) for _ in range(NBANK)]
            + [pltpu.VMEM((C, D), jnp.float32) for _ in range(NBANK)]
            + [pltpu.SemaphoreType.DMA for _ in range(1 + 3 * NBANK)]
        ),
    )(_sc_body)
    return kern(ent_x, ent_a, meta, xf, af)


def kernel(x, attack, attack_mask):
    xf = x.reshape(N, D)
    af = attack.reshape(N, D)
    ent_x, ent_a, meta = _prep(attack_mask)
    out = _sc_select(xf, af, ent_x, ent_a, meta)
    return out.reshape(B, S, D)


# speedup vs baseline: 1.8791x; 1.4833x over previous
import functools

import jax
import jax.numpy as jnp
from jax import lax
from jax.experimental import pallas as pl
from jax.experimental.pallas import tpu as pltpu
from jax.experimental.pallas import tpu_sc as plsc

B, S, D = 4, 4096, 2048
N = B * S
NW = 32
SEG = N // NW
C = 16
NCH = SEG // C
TMAX = NCH + 1


def _prep(attack_mask):
    m = attack_mask.reshape(NW, SEG).astype(jnp.int32)
    p = jnp.arange(SEG, dtype=jnp.int32)[None, :]
    nxw = SEG - jnp.sum(m, axis=1, keepdims=True)
    order = jnp.argsort(m, axis=1, stable=True).astype(jnp.int32)
    permg = order + (jnp.arange(NW, dtype=jnp.int32) * SEG)[:, None]
    lastx = jnp.take_along_axis(permg, jnp.maximum(nxw - 1, 0), axis=1)
    firsta = jnp.take_along_axis(permg, jnp.minimum(nxw, SEG - 1), axis=1)
    ent_x = jnp.where(p >= nxw, lastx, permg).reshape(-1)
    ent_a = jnp.where(p < nxw, firsta, permg).reshape(-1)
    meta = nxw.reshape(2, 16)
    return ent_x, ent_a, meta


NBANK = 3
NGRP = (TMAX + NBANK - 1) // NBANK


def _sc_body(entx, enta, meta, xf, af, out,
             meta_v, ib0, ib1, ib2, b0, b1, b2,
             msem, is0, is1, is2, gs0, gs1, gs2, ss0, ss1, ss2):
    ibufs = (ib0, ib1, ib2)
    bufs = (b0, b1, b2)
    isems = (is0, is1, is2)
    gsems = (gs0, gs1, gs2)
    ssems = (ss0, ss1, ss2)

    c = lax.axis_index("c")
    s = lax.axis_index("s")
    wid = c * 16 + s
    mcp = pltpu.make_async_copy(meta.at[c], meta_v, msem)
    mcp.start()
    mcp.wait()
    mv = meta_v[...]
    nxw = jnp.int32(0)
    for k in range(16):
        nxw = nxw + jnp.where(s == k, mv[k], 0)
    n_xc = (nxw + C - 1) // C
    t = n_xc + (NCH - nxw // C)
    base = wid * SEG

    def ent_off(i):
        is_x = i < n_xc
        chunk = jnp.where(is_x, i, nxw // C + (i - n_xc))
        return is_x, base + chunk * C

    def start_idx(i, bank):
        is_x, off = ent_off(i)

        @pl.when(is_x)
        def _():
            pltpu.make_async_copy(
                entx.at[pl.ds(off, C)], ibufs[bank], isems[bank]).start()

        @pl.when(jnp.logical_not(is_x))
        def _():
            pltpu.make_async_copy(
                enta.at[pl.ds(off, C)], ibufs[bank], isems[bank]).start()

    start_idx(0, 0)

    def group(g, carry):
        for b in range(NBANK):
            i = g * NBANK + b
            nb = (b + 1) % NBANK
            pb = (b + 2) % NBANK

            @pl.when(i < t)
            def _(i=i, b=b, nb=nb, pb=pb):
                pltpu.make_async_copy(
                    entx.at[pl.ds(0, C)], ibufs[b], isems[b]).wait()

                @pl.when(i >= 2)
                def _():
                    pltpu.make_async_copy(
                        bufs[nb], out.at[ibufs[nb]], ssems[nb]).wait()

                @pl.when(i + 1 < t)
                def _():
                    start_idx(i + 1, nb)

                is_x, _ = ent_off(i)

                @pl.when(is_x)
                def _():
                    pltpu.make_async_copy(
                        xf.at[ibufs[b]], bufs[b], gsems[b]).start()

                @pl.when(jnp.logical_not(is_x))
                def _():
                    pltpu.make_async_copy(
                        af.at[ibufs[b]], bufs[b], gsems[b]).start()

                @pl.when(i >= 1)
                def _():
                    pltpu.make_async_copy(
                        xf.at[ibufs[pb]], bufs[pb], gsems[pb]).wait()
                    pltpu.make_async_copy(
                        bufs[pb], out.at[ibufs[pb]], ssems[pb]).start()
        return carry

    lax.fori_loop(0, NGRP, group, 0)

    for b in range(NBANK):
        @pl.when((t - 1) % NBANK == b)
        def _(b=b):
            pltpu.make_async_copy(xf.at[ibufs[b]], bufs[b], gsems[b]).wait()
            pltpu.make_async_copy(bufs[b], out.at[ibufs[b]], ssems[b]).start()
    for b in range(NBANK):
        @pl.when(jnp.logical_or((t - 1) % NBANK == b, (t - 2) % NBANK == b))
        def _(b=b):
            pltpu.make_async_copy(bufs[b], out.at[ibufs[b]], ssems[b]).wait()


@jax.jit
def _sc_select(xf, af, ent_x, ent_a, meta):
    mesh = plsc.VectorSubcoreMesh(core_axis_name="c", subcore_axis_name="s")
    kern = functools.partial(
        pl.kernel,
        out_type=jax.ShapeDtypeStruct((N, D), jnp.float32),
        mesh=mesh,
        scratch_types=(
            [pltpu.VMEM((16,), jnp.int32)]
            + [pltpu.VMEM((C,), jnp.int32) for _ in range(NBANK)]
            + [pltpu.VMEM((C, D), jnp.float32) for _ in range(NBANK)]
            + [pltpu.SemaphoreType.DMA for _ in range(1 + 3 * NBANK)]
        ),
    )(_sc_body)
    return kern(ent_x, ent_a, meta, xf, af)


def kernel(x, attack, attack_mask):
    xf = x.reshape(N, D)
    af = attack.reshape(N, D)
    ent_x, ent_a, meta = _prep(attack_mask)
    out = _sc_select(xf, af, ent_x, ent_a, meta)
    return out.reshape(B, S, D)
